# baseline (speedup 1.0000x reference)
import jax
import jax.numpy as jnp
from jax.experimental import pallas as pl
from jax.experimental.pallas import tpu as pltpu

_TILE = 64


def _make_gather_kernel(sub):
    def _gather_kernel(idx_ref, w_ref, o_ref):
        base = pl.program_id(0) * _TILE
        for mi in range(_TILE):
            i = pl.multiple_of(idx_ref[base + mi], sub)
            o_ref[pl.ds(mi * sub, sub), :] = w_ref[pl.ds(i, sub), :]

    return _gather_kernel


def _round_up(n, m):
    return ((n + m - 1) // m) * m


def kernel(x, weight):
    B, S = x.shape
    V, D = weight.shape
    N = B * S

    D_pad = _round_up(D, 128)
    if D_pad != D:
        weight = jnp.pad(weight, ((0, 0), (0, D_pad - D)))
    sub = D_pad // 128

    idx = jnp.clip(x.reshape(N).astype(jnp.int32), 0, V - 1)
    N_pad = _round_up(N, _TILE)
    if N_pad != N:
        idx = jnp.pad(idx, (0, N_pad - N))
    idx = idx * sub

    out = pl.pallas_call(
        _make_gather_kernel(sub),
        out_shape=jax.ShapeDtypeStruct((N_pad * sub, 128), weight.dtype),
        grid_spec=pltpu.PrefetchScalarGridSpec(
            num_scalar_prefetch=1,
            grid=(N_pad // _TILE,),
            in_specs=[
                pl.BlockSpec((V * sub, 128), lambda i, ids: (0, 0)),
            ],
            out_specs=pl.BlockSpec((_TILE * sub, 128), lambda i, ids: (i, 0)),
        ),
        compiler_params=pltpu.CompilerParams(
            dimension_semantics=("parallel",),
            vmem_limit_bytes=48 * 1024 * 1024,
        ),
    )(idx, weight.reshape(V * sub, 128))

    return out.reshape(N_pad, D_pad)[:N, :D].reshape(B, S, D)


# speedup vs baseline: 1.9437x; 1.9437x over previous
import jax
import jax.numpy as jnp
from jax.experimental import pallas as pl
from jax.experimental.pallas import tpu as pltpu

_TILE = 512
_UNROLL = 64


def _make_gather_kernel(sub):
    def _gather_kernel(idx_ref, w_ref, o_ref):
        base = pl.program_id(0) * _TILE

        def chunk(c, _):
            off = c * _UNROLL
            for mi in range(_UNROLL):
                i = pl.multiple_of(idx_ref[base + off + mi], sub)
                o = pl.multiple_of((off + mi) * sub, sub)
                o_ref[pl.ds(o, sub), :] = w_ref[pl.ds(i, sub), :]
            return _

        jax.lax.fori_loop(0, _TILE // _UNROLL, chunk, 0)

    return _gather_kernel


def _round_up(n, m):
    return ((n + m - 1) // m) * m


def kernel(x, weight):
    B, S = x.shape
    V, D = weight.shape
    N = B * S

    D_pad = _round_up(D, 128)
    if D_pad != D:
        weight = jnp.pad(weight, ((0, 0), (0, D_pad - D)))
    sub = D_pad // 128

    idx = jnp.clip(x.reshape(N).astype(jnp.int32), 0, V - 1)
    N_pad = _round_up(N, _TILE)
    if N_pad != N:
        idx = jnp.pad(idx, (0, N_pad - N))
    idx = idx * sub

    out = pl.pallas_call(
        _make_gather_kernel(sub),
        out_shape=jax.ShapeDtypeStruct((N_pad * sub, 128), weight.dtype),
        grid_spec=pltpu.PrefetchScalarGridSpec(
            num_scalar_prefetch=1,
            grid=(N_pad // _TILE,),
            in_specs=[
                pl.BlockSpec((V * sub, 128), lambda i, ids: (0, 0)),
            ],
            out_specs=pl.BlockSpec((_TILE * sub, 128), lambda i, ids: (i, 0)),
        ),
        compiler_params=pltpu.CompilerParams(
            dimension_semantics=("parallel",),
            vmem_limit_bytes=48 * 1024 * 1024,
        ),
    )(idx, weight.reshape(V * sub, 128))

    return out.reshape(N_pad, D_pad)[:N, :D].reshape(B, S, D)


# speedup vs baseline: 2.5848x; 1.3299x over previous
import jax
import jax.numpy as jnp
from jax.experimental import pallas as pl
from jax.experimental.pallas import tpu as pltpu

_TILE = 512
_UNROLL = 32


def _make_gather_kernel(d_pad):
    def _gather_kernel(chunk_ref, shift_ref, w_ref, o_ref):
        iota8 = jax.lax.broadcasted_iota(jnp.int32, (8, d_pad), 0)
        base = pl.program_id(0) * _TILE

        def body(c, _):
            off = c * _UNROLL
            for g in range(_UNROLL // 8):
                goff = off + g * 8
                acc = None
                for t in range(8):
                    n = base + goff + t
                    c8 = pl.multiple_of(chunk_ref[n], 8)
                    chunk = w_ref[pl.ds(c8, 8), :]
                    rot = pltpu.roll(chunk, shift_ref[n], axis=0)
                    acc = rot if t == 0 else jnp.where(iota8 == t, rot, acc)
                o_ref[pl.ds(pl.multiple_of(goff, 8), 8), :] = acc
            return _

        jax.lax.fori_loop(0, _TILE // _UNROLL, body, 0)

    return _gather_kernel


def _round_up(n, m):
    return ((n + m - 1) // m) * m


def kernel(x, weight):
    B, S = x.shape
    V, D = weight.shape
    N = B * S

    D_pad = _round_up(D, 128)
    if D_pad != D:
        weight = jnp.pad(weight, ((0, 0), (0, D_pad - D)))

    idx = jnp.clip(x.reshape(N).astype(jnp.int32), 0, V - 1)
    N_pad = _round_up(N, _TILE)
    if N_pad != N:
        idx = jnp.pad(idx, (0, N_pad - N))

    pos = jax.lax.iota(jnp.int32, N_pad)
    chunk_base = (idx >> 3) << 3
    shift = ((pos & 7) - (idx & 7)) & 7

    out = pl.pallas_call(
        _make_gather_kernel(D_pad),
        out_shape=jax.ShapeDtypeStruct((N_pad, D_pad), weight.dtype),
        grid_spec=pltpu.PrefetchScalarGridSpec(
            num_scalar_prefetch=2,
            grid=(N_pad // _TILE,),
            in_specs=[
                pl.BlockSpec((V, D_pad), lambda i, cb, sh: (0, 0)),
            ],
            out_specs=pl.BlockSpec((_TILE, D_pad), lambda i, cb, sh: (i, 0)),
        ),
        compiler_params=pltpu.CompilerParams(
            dimension_semantics=("parallel",),
            vmem_limit_bytes=48 * 1024 * 1024,
        ),
    )(chunk_base, shift, weight)

    return out[:N, :D].reshape(B, S, D)
